# baseline (speedup 1.0000x reference)
import jax
import jax.numpy as jnp
from jax.experimental import pallas as pl

F_IN = 128
F_H = 32
F_OUT = 2
N_ROWS = 10000
BLK = 1000


def _body(x_ref, wz0_ref, wz1_ref, bz_ref, wh0_ref, wh1_ref, bh_ref,
          wlin_ref, blin_ref, out_ref, h_ref):
    xb = x_ref[...]
    wz = wz0_ref[...] + wz1_ref[...]
    wh = wh0_ref[...] + wh1_ref[...]
    z = jax.nn.sigmoid(
        jnp.dot(xb, wz, preferred_element_type=jnp.float32) + bz_ref[...])
    ht = jnp.tanh(
        jnp.dot(xb, wh, preferred_element_type=jnp.float32) + bh_ref[...])
    hn = (1.0 - z) * ht
    h_ref[...] = hn
    out_ref[...] = jax.lax.dot_general(
        jnp.maximum(hn, 0.0), wlin_ref[...],
        dimension_numbers=(((1,), (1,)), ((), ())),
        preferred_element_type=jnp.float32) + blin_ref[...]


def kernel(x, edge_index, edge_weight, W_z, b_z, W_r, b_r, W_h, b_h,
           W_lin, b_lin):
    del edge_index, edge_weight, W_r, b_r
    wz0 = W_z[0, 0, :F_IN]
    wz1 = W_z[1, 0, :F_IN]
    wh0 = W_h[0, 0, :F_IN]
    wh1 = W_h[1, 0, :F_IN]
    bz = b_z.reshape(1, F_H)
    bh = b_h.reshape(1, F_H)
    blin = b_lin.reshape(1, F_OUT)

    grid = (N_ROWS // BLK,)
    full = lambda shape: pl.BlockSpec(shape, lambda i: (0, 0))
    out, h_new = pl.pallas_call(
        _body,
        grid=grid,
        in_specs=[
            pl.BlockSpec((BLK, F_IN), lambda i: (i, 0)),
            full((F_IN, F_H)), full((F_IN, F_H)), full((1, F_H)),
            full((F_IN, F_H)), full((F_IN, F_H)), full((1, F_H)),
            full((F_OUT, F_H)), full((1, F_OUT)),
        ],
        out_specs=[
            pl.BlockSpec((BLK, F_OUT), lambda i: (i, 0)),
            pl.BlockSpec((BLK, F_H), lambda i: (i, 0)),
        ],
        out_shape=[
            jax.ShapeDtypeStruct((N_ROWS, F_OUT), jnp.float32),
            jax.ShapeDtypeStruct((N_ROWS, F_H), jnp.float32),
        ],
    )(x, wz0, wz1, bz, wh0, wh1, bh, W_lin, blin)
    return (out, h_new)


# speedup vs baseline: 1.1370x; 1.1370x over previous
import jax
import jax.numpy as jnp
from jax.experimental import pallas as pl

F_IN = 128
F_H = 32
F_OUT = 2
N_ROWS = 10000
BLK = 2000


def _body(x_ref, wz0_ref, wz1_ref, bz_ref, wh0_ref, wh1_ref, bh_ref,
          wlin_ref, blin_ref, out_ref, h_ref):
    xb = x_ref[...]
    wz = wz0_ref[...] + wz1_ref[...]
    wh = wh0_ref[...] + wh1_ref[...]
    z = jax.nn.sigmoid(
        jnp.dot(xb, wz, preferred_element_type=jnp.float32) + bz_ref[...])
    ht = jnp.tanh(
        jnp.dot(xb, wh, preferred_element_type=jnp.float32) + bh_ref[...])
    hn = (1.0 - z) * ht
    h_ref[...] = hn
    out_ref[...] = jax.lax.dot_general(
        jnp.maximum(hn, 0.0), wlin_ref[...],
        dimension_numbers=(((1,), (1,)), ((), ())),
        preferred_element_type=jnp.float32) + blin_ref[...]


def kernel(x, edge_index, edge_weight, W_z, b_z, W_r, b_r, W_h, b_h,
           W_lin, b_lin):
    del edge_index, edge_weight, W_r, b_r
    wz0 = W_z[0, 0, :F_IN]
    wz1 = W_z[1, 0, :F_IN]
    wh0 = W_h[0, 0, :F_IN]
    wh1 = W_h[1, 0, :F_IN]
    bz = b_z.reshape(1, F_H)
    bh = b_h.reshape(1, F_H)
    blin = b_lin.reshape(1, F_OUT)

    grid = (N_ROWS // BLK,)
    full = lambda shape: pl.BlockSpec(shape, lambda i: (0, 0))
    out, h_new = pl.pallas_call(
        _body,
        grid=grid,
        in_specs=[
            pl.BlockSpec((BLK, F_IN), lambda i: (i, 0)),
            full((F_IN, F_H)), full((F_IN, F_H)), full((1, F_H)),
            full((F_IN, F_H)), full((F_IN, F_H)), full((1, F_H)),
            full((F_OUT, F_H)), full((1, F_OUT)),
        ],
        out_specs=[
            pl.BlockSpec((BLK, F_OUT), lambda i: (i, 0)),
            pl.BlockSpec((BLK, F_H), lambda i: (i, 0)),
        ],
        out_shape=[
            jax.ShapeDtypeStruct((N_ROWS, F_OUT), jnp.float32),
            jax.ShapeDtypeStruct((N_ROWS, F_H), jnp.float32),
        ],
    )(x, wz0, wz1, bz, wh0, wh1, bh, W_lin, blin)
    return (out, h_new)


# speedup vs baseline: 1.1689x; 1.0281x over previous
import jax
import jax.numpy as jnp
from jax.experimental import pallas as pl

F_IN = 128
F_H = 32
F_OUT = 2
N_ROWS = 10000
BLK = 5000


def _body(x_ref, wz0_ref, wz1_ref, bz_ref, wh0_ref, wh1_ref, bh_ref,
          wlin_ref, blin_ref, out_ref, h_ref):
    xb = x_ref[...]
    wz = wz0_ref[...] + wz1_ref[...]
    wh = wh0_ref[...] + wh1_ref[...]
    z = jax.nn.sigmoid(
        jnp.dot(xb, wz, preferred_element_type=jnp.float32) + bz_ref[...])
    ht = jnp.tanh(
        jnp.dot(xb, wh, preferred_element_type=jnp.float32) + bh_ref[...])
    hn = (1.0 - z) * ht
    h_ref[...] = hn
    out_ref[...] = jax.lax.dot_general(
        jnp.maximum(hn, 0.0), wlin_ref[...],
        dimension_numbers=(((1,), (1,)), ((), ())),
        preferred_element_type=jnp.float32) + blin_ref[...]


def kernel(x, edge_index, edge_weight, W_z, b_z, W_r, b_r, W_h, b_h,
           W_lin, b_lin):
    del edge_index, edge_weight, W_r, b_r
    wz0 = W_z[0, 0, :F_IN]
    wz1 = W_z[1, 0, :F_IN]
    wh0 = W_h[0, 0, :F_IN]
    wh1 = W_h[1, 0, :F_IN]
    bz = b_z.reshape(1, F_H)
    bh = b_h.reshape(1, F_H)
    blin = b_lin.reshape(1, F_OUT)

    grid = (N_ROWS // BLK,)
    full = lambda shape: pl.BlockSpec(shape, lambda i: (0, 0))
    out, h_new = pl.pallas_call(
        _body,
        grid=grid,
        in_specs=[
            pl.BlockSpec((BLK, F_IN), lambda i: (i, 0)),
            full((F_IN, F_H)), full((F_IN, F_H)), full((1, F_H)),
            full((F_IN, F_H)), full((F_IN, F_H)), full((1, F_H)),
            full((F_OUT, F_H)), full((1, F_OUT)),
        ],
        out_specs=[
            pl.BlockSpec((BLK, F_OUT), lambda i: (i, 0)),
            pl.BlockSpec((BLK, F_H), lambda i: (i, 0)),
        ],
        out_shape=[
            jax.ShapeDtypeStruct((N_ROWS, F_OUT), jnp.float32),
            jax.ShapeDtypeStruct((N_ROWS, F_H), jnp.float32),
        ],
    )(x, wz0, wz1, bz, wh0, wh1, bh, W_lin, blin)
    return (out, h_new)
